# baseline (speedup 1.0000x reference)
import jax
import jax.numpy as jnp
from jax.experimental import pallas as pl
from jax.experimental.pallas import tpu as pltpu

_B, _F, _NC, _P = 8, 256, 1024, 576
_NBS = 8
_NSTEP = _B // _NBS


def _tc_body(x_ref, c_ref, y_ref, loss_ref):
    g = pl.program_id(0)
    C = c_ref[...]
    c_sq = jnp.sum(C * C, axis=0, keepdims=True)
    iota0 = jax.lax.broadcasted_iota(jnp.int32, (_NC, _P), 0)
    partial = jnp.float32(0.0)
    for i in range(_NBS):
        xb = x_ref[i]
        S2 = jax.lax.dot_general(C, xb, (((0,), (0,)), ((), ())),
                                 preferred_element_type=jnp.float32)
        score = c_sq.T - 2.0 * S2
        idx = jnp.argmin(score, axis=0)
        oh = (iota0 == idx[None, :]).astype(jnp.float32)
        xq = jax.lax.dot_general(C, oh, (((1,), (0,)), ((), ())),
                                 preferred_element_type=jnp.float32)
        y_ref[i] = xq
        r = xb - xq
        partial += jnp.sum(r * r)

    @pl.when(g == 0)
    def _():
        loss_ref[0, 0] = 0.0

    loss_ref[0, 0] += partial

    @pl.when(g == _NSTEP - 1)
    def _():
        loss_ref[0, 0] = loss_ref[0, 0] / (_B * _F * _P)


def kernel(x, centroids):
    x3 = x.reshape(_B, _F, _P)
    y, loss = pl.pallas_call(
        _tc_body,
        grid=(_NSTEP,),
        in_specs=[
            pl.BlockSpec((_NBS, _F, _P), lambda g: (g, 0, 0)),
            pl.BlockSpec((_F, _NC), lambda g: (0, 0)),
        ],
        out_specs=[
            pl.BlockSpec((_NBS, _F, _P), lambda g: (g, 0, 0)),
            pl.BlockSpec(memory_space=pltpu.SMEM, block_shape=(1, 1),
                         index_map=lambda g: (0, 0)),
        ],
        out_shape=[
            jax.ShapeDtypeStruct((_B, _F, _P), jnp.float32),
            jax.ShapeDtypeStruct((1, 1), jnp.float32),
        ],
        compiler_params=pltpu.CompilerParams(
            dimension_semantics=("arbitrary",),
        ),
    )(x3, centroids)
    return y.reshape(_B, _F, 24, 24), loss[0, 0]
